# Initial kernel scaffold; baseline (speedup 1.0000x reference)
import jax
import jax.numpy as jnp
from jax.experimental import pallas as pl


def kernel(data_NxSxA, data_batched_NxSxA, site_positions_batched_SxSfull, W_enc, W_sitepos, W_merge, W_d, W_c, W_rate, W_b):
    raise NotImplementedError("write your pallas kernel here")


# speedup vs baseline: 6.1150x; 6.1150x over previous
import jax
import jax.numpy as jnp
from jax import lax
from jax.experimental import pallas as pl
from jax.experimental.pallas import tpu as pltpu

_K = 128
_PRIOR_BRANCH_LEN = 0.1


def _body(embl_ref, lfT_ref, stT_ref, Wm_ref, Wd_ref,
          Wr_ref, Wb_ref, g1_ref, g2_ref, g3_ref, cpr_ref, cc_ref,
          o_logZ, o_ll, o_m1, o_m2, o_b1, o_b2, o_embb,
          h_idx, h_i1, h_i2, h_b1, h_lw, h_ll,
          fels_s, emb_s, h_b2, h_emb):
    f32 = jnp.float32
    N, S, A, D, K = 24, 128, 4, 64, _K
    R = N - 1

    def fiota(shape, dim):
        return lax.broadcasted_iota(jnp.int32, shape, dim).astype(f32)

    ioK_r = fiota((1, K), 1)
    ioK_c = fiota((K, 1), 0)
    eyeK = (lax.broadcasted_iota(jnp.int32, (K, K), 0)
            == lax.broadcasted_iota(jnp.int32, (K, K), 1)).astype(f32)
    eyeN = (lax.broadcasted_iota(jnp.int32, (N, N), 0)
            == lax.broadcasted_iota(jnp.int32, (N, N), 1)).astype(f32)

    def tr_row(col):
        return jnp.sum(eyeK * col, axis=0, keepdims=True)

    def amax_lanes(v, L):
        mx = jnp.max(v, axis=1, keepdims=True)
        io = fiota(v.shape, 1)
        idxf = jnp.min(jnp.where(v == mx, io, f32(L)), axis=1, keepdims=True)
        oh = (io == idxf).astype(f32)
        return idxf, oh

    def max4(ts):
        return jnp.maximum(jnp.maximum(ts[0], ts[1]), jnp.maximum(ts[2], ts[3]))

    def sum4(ts):
        return (ts[0] + ts[2]) + (ts[1] + ts[3])

    def sum_minor(x):
        n = x.shape[1]
        t = x[:, 0:8]
        for r in range(1, n // 8):
            t = t + x[:, 8 * r:8 * r + 8]
        u = t[:, 0:4] + t[:, 4:8]
        v = u[:, 0:2] + u[:, 2:4]
        return v[:, 0:1] + v[:, 1:2]

    def lsm4(ts):
        m = max4(ts)
        sh = [t - m for t in ts]
        ls = jnp.log(sum4([jnp.exp(x) for x in sh]))
        return [x - ls for x in sh]

    def lse4(ts):
        m = max4(ts)
        return jnp.log(sum4([jnp.exp(t - m) for t in ts])) + m

    def laddexp(x1, x2):
        return jnp.maximum(x1, x2) + jnp.log1p(jnp.exp(-jnp.abs(x1 - x2)))

    def rd(ref, j):
        return ref[pl.ds(j, 1)][0]

    def wr(ref, j, x):
        ref[pl.ds(j, 1)] = x[None]

    k10 = cc_ref[0:1, 0:1]
    lgK = cc_ref[0:1, 1:2]

    stT = [stT_ref[a:a + 1, :] for a in range(A)]

    emb_leaf = embl_ref[:, :]
    z_leaf = jnp.dot(emb_leaf, Wd_ref[:, :])
    leaf_lf = [lfT_ref[a] for a in range(A)]
    lps_leaf = lsm4([z_leaf[:, a:a + 1] + stT[a] for a in range(A)])
    per_site0 = lse4([lps_leaf[a] + leaf_lf[a] for a in range(A)])
    pn0_col = sum_minor(per_site0)
    pn0_row = jnp.sum(eyeN * pn0_col, axis=0, keepdims=True)

    for n in range(N):
        wr(emb_s, n, jnp.broadcast_to(emb_leaf[n:n + 1, :], (K, D)))
        lf_n = jnp.concatenate([leaf_lf[a][n:n + 1, :] for a in range(A)], axis=1)
        wr(fels_s, n, jnp.broadcast_to(lf_n, (K, S * A)))

    alive0 = jnp.ones((K, N), f32)
    per_node0 = jnp.broadcast_to(pn0_row, (K, N))
    zcol = jnp.zeros((K, 1), f32)
    lw0 = jnp.broadcast_to(0.0 - lgK, (K, 1))

    def round_body(r, carry):
        alive, per_node, log_pi, log_prior, log_weight, logZ, log_lik = carry
        p = lax.rem(r, 2)
        bs = p * N
        bd = (1 - p) * N

        g1r = rd(g1_ref, r)
        lw_row = tr_row(log_weight)
        idx_col, ohK = amax_lanes(g1r + lw_row, K)

        alive = jnp.dot(ohK, alive)
        per_node = jnp.dot(ohK, per_node)
        log_pi = jnp.dot(ohK, log_pi)
        log_prior = jnp.dot(ohK, log_prior)

        g2r = rd(g2_ref, r)
        la = jnp.log(alive + 1e-20)
        i1_col, oh1 = amax_lanes(g2r + la, N)
        alive_minus = alive * (1.0 - oh1)
        g3r = rd(g3_ref, r)
        lam = jnp.log(alive_minus + 1e-20)
        i2_col, oh2 = amax_lanes(g3r + lam, N)

        e1 = jnp.zeros((K, D), f32)
        e2 = jnp.zeros((K, D), f32)
        f1 = jnp.zeros((K, S * A), f32)
        f2 = jnp.zeros((K, S * A), f32)
        for n in range(N):
            frow = jnp.dot(ohK, rd(fels_s, bs + n))
            erow = jnp.dot(ohK, rd(emb_s, bs + n))
            wr(fels_s, bd + n, frow)
            wr(emb_s, bd + n, erow)
            m1n = oh1[:, n:n + 1]
            m2n = oh2[:, n:n + 1]
            f1 = f1 + m1n * frow
            f2 = f2 + m2n * frow
            e1 = e1 + m1n * erow
            e2 = e2 + m2n * erow

        pair = jnp.concatenate([e1, e2], axis=1)
        new_emb = jnp.tanh(jnp.dot(pair, Wm_ref[:, :]))
        blz = jnp.dot(pair, Wb_ref[:, :])
        bl = (jnp.maximum(blz, 0.0) + jnp.log1p(jnp.exp(-jnp.abs(blz)))) + 1e-4
        b1 = bl[:, 0:1]
        b2 = bl[:, 1:2]

        z = jnp.dot(new_emb, Wd_ref[:, :])
        murz = jnp.dot(new_emb, Wr_ref[:, :])
        mur = (jnp.maximum(murz, 0.0) + jnp.log1p(jnp.exp(-jnp.abs(murz)))) + 1e-3
        lps = lsm4([z[:, a:a + 1] + stT[a] for a in range(A)])

        f1a = [f1[:, a * S:(a + 1) * S] for a in range(A)]
        f2a = [f2[:, a * S:(a + 1) * S] for a in range(A)]

        def propagate(lfa, br):
            loge = -(mur * br)
            l1me = jnp.log1p(-jnp.exp(loge) + 1e-12)
            mix = lse4([lps[a] + lfa[a] for a in range(A)])
            return [laddexp(loge + lfa[a], l1me + mix) for a in range(A)]

        p1 = propagate(f1a, b1)
        p2 = propagate(f2a, b2)
        nf = [p1[a] + p2[a] for a in range(A)]
        nf_cat = jnp.concatenate(nf, axis=1)

        per_site = lse4([lps[a] + nf[a] for a in range(A)])
        pn_new = sum_minor(per_site)

        for n in range(N):
            m1n = oh1[:, n:n + 1]
            fd = rd(fels_s, bd + n)
            wr(fels_s, bd + n, fd * (1.0 - m1n) + nf_cat * m1n)
            ed = rd(emb_s, bd + n)
            wr(emb_s, bd + n, ed * (1.0 - m1n) + new_emb * m1n)
        alive = alive * (1.0 - oh2)
        per_node = per_node * (1.0 - oh1) + pn_new * oh1

        new_log_lik = sum_minor(per_node * alive)
        lp1 = k10 - 10.0 * b1
        lp2 = k10 - 10.0 * b2
        log_prior = (log_prior + lp1) + lp2
        new_log_pi = new_log_lik + log_prior
        c1 = cpr_ref[pl.ds(r, 1), 0:1]
        log_v_plus = (c1 + (0.0 - (1.0 * b1))) + (0.0 - (1.0 * b2))
        log_weight = (new_log_pi - log_pi) - log_v_plus
        mw = jnp.max(log_weight, axis=0, keepdims=True)
        se = jnp.sum(jnp.exp(log_weight - mw), axis=0, keepdims=True)
        logZ = (logZ + (jnp.log(se) + mw)) - lgK
        log_pi = new_log_pi
        log_lik = new_log_lik

        h_idx[pl.ds(r, 1), :] = tr_row(idx_col)
        h_i1[pl.ds(r, 1), :] = tr_row(i1_col)
        h_i2[pl.ds(r, 1), :] = tr_row(i2_col)
        h_b1[pl.ds(r, 1), :] = tr_row(b1)
        h_b2[pl.ds(r, 1), :] = tr_row(b2)
        h_lw[pl.ds(r, 1), :] = tr_row(log_weight)
        h_ll[pl.ds(r, 1), :] = tr_row(new_log_lik)
        wr(h_emb, r, new_emb)

        return (alive, per_node, log_pi, log_prior, log_weight, logZ, log_lik)

    carry = (alive0, per_node0, zcol, zcol, lw0, jnp.zeros((1, 1), f32), zcol)
    carry = lax.fori_loop(0, R, round_body, carry)
    (_, _, _, _, _, logZ, log_lik) = carry

    o_logZ[:, :] = logZ
    o_ll[:, :] = log_lik

    mx = jnp.max(log_lik, axis=0, keepdims=True)
    best = jnp.min(jnp.where(log_lik == mx, ioK_c, f32(K)), axis=0, keepdims=True)

    def bt_body(t, val):
        r = (R - 1) - t
        ohr = (ioK_r == val).astype(f32)
        ohc = (ioK_c == val).astype(f32)
        o_m1[pl.ds(r, 1), :] = jnp.sum(h_i1[pl.ds(r, 1), :] * ohr, axis=1,
                                       keepdims=True)
        o_m2[pl.ds(r, 1), :] = jnp.sum(h_i2[pl.ds(r, 1), :] * ohr, axis=1,
                                       keepdims=True)
        o_b1[pl.ds(r, 1), :] = jnp.sum(h_b1[pl.ds(r, 1), :] * ohr, axis=1,
                                       keepdims=True)
        o_b2[pl.ds(r, 1), :] = jnp.sum(h_b2[pl.ds(r, 1), :] * ohr, axis=1,
                                       keepdims=True)
        o_embb[pl.ds(r, 1), :] = jnp.sum(rd(h_emb, r) * ohc, axis=0,
                                         keepdims=True)
        return jnp.sum(h_idx[pl.ds(r, 1), :] * ohr, axis=1, keepdims=True)

    lax.fori_loop(0, R, bt_body, best)


def kernel(data_NxSxA, data_batched_NxSxA, site_positions_batched_SxSfull,
           W_enc, W_sitepos, W_merge, W_d, W_c, W_rate, W_b):
    f32 = jnp.float32
    N, Sf, A = data_NxSxA.shape
    _, S, _ = data_batched_NxSxA.shape
    D = W_enc.shape[1]
    R = N - 1
    K = _K

    key = jax.random.key(42)
    g1l, g2l, g3l = [], [], []
    for _ in range(R):
        key, k1, k2, k3 = jax.random.split(key, 4)
        g1l.append(jax.random.gumbel(k1, (K, K), f32))
        g2l.append(jax.random.gumbel(k2, (K, N), f32))
        g3l.append(jax.random.gumbel(k3, (K, N), f32))
    g1 = jnp.stack(g1l)
    g2 = jnp.stack(g2l)
    g3 = jnp.stack(g3l)

    cpr_l = []
    for r in range(R):
        t_alive = float(N - r)
        n_pairs = t_alive * (t_alive - 1.0) / 2.0
        cpr_l.append(-jnp.log(n_pairs))
    cpr = jnp.broadcast_to(jnp.stack(cpr_l).astype(f32).reshape(R, 1), (R, 128))
    prior_rate = 1.0 / _PRIOR_BRANCH_LEN
    z0 = jnp.float32(0.0)
    cc = jnp.stack([jnp.log(prior_rate).astype(f32),
                    jnp.log(float(K)).astype(f32),
                    z0, z0, z0, z0, z0, z0]).reshape(1, 8)

    site_SxC = jnp.tanh(site_positions_batched_SxSfull @ W_sitepos)
    stT = jnp.transpose(site_SxC @ W_c)
    emb_leaf = jnp.tanh(data_NxSxA.reshape(N, -1) @ W_enc)
    lfT = jnp.transpose(jnp.log(data_batched_NxSxA), (2, 0, 1))

    out_shape = [
        jax.ShapeDtypeStruct((1, 1), f32),
        jax.ShapeDtypeStruct((K, 1), f32),
        jax.ShapeDtypeStruct((R, 1), f32),
        jax.ShapeDtypeStruct((R, 1), f32),
        jax.ShapeDtypeStruct((R, 1), f32),
        jax.ShapeDtypeStruct((R, 1), f32),
        jax.ShapeDtypeStruct((R, D), f32),
        jax.ShapeDtypeStruct((R, K), f32),
        jax.ShapeDtypeStruct((R, K), f32),
        jax.ShapeDtypeStruct((R, K), f32),
        jax.ShapeDtypeStruct((R, K), f32),
        jax.ShapeDtypeStruct((R, K), f32),
        jax.ShapeDtypeStruct((R, K), f32),
    ]
    scratch_shapes = [
        pltpu.VMEM((2 * N, K, S * A), f32),
        pltpu.VMEM((2 * N, K, D), f32),
        pltpu.VMEM((R, K), f32),
        pltpu.VMEM((R, K, D), f32),
    ]
    outs = pl.pallas_call(
        _body,
        out_shape=out_shape,
        scratch_shapes=scratch_shapes,
    )(emb_leaf, lfT, stT, W_merge, W_d, W_rate, W_b, g1, g2, g3, cpr, cc)

    o_logZ, o_ll, o_m1, o_m2, o_b1, o_b2, o_embb = outs[:7]
    log_Z = o_logZ[0, 0]
    return (-log_Z, log_Z, o_ll[:, 0],
            o_m1[:, 0].astype(jnp.int32), o_m2[:, 0].astype(jnp.int32),
            o_b1[:, 0], o_b2[:, 0], o_embb)
